# Initial kernel scaffold; baseline (speedup 1.0000x reference)
import jax
import jax.numpy as jnp
from jax.experimental import pallas as pl


def kernel(seed_xyz, seed_features, Wg1, bg1, gn_gamma, gn_beta, gn_mean, gn_var, Wg2, bg2, W1, b1, bn1_gamma, bn1_beta, bn1_mean, bn1_var, W2, b2):
    raise NotImplementedError("write your pallas kernel here")


# speedup vs baseline: 20.0699x; 20.0699x over previous
import functools

import numpy as np
import jax
import jax.numpy as jnp
from jax import lax
from jax.experimental import pallas as pl
from jax.experimental.pallas import tpu as pltpu
from jax.experimental.pallas import tpu_sc as plsc

_B, _N, _C, _M, _V = 4, 20000, 256, 1024, 300
_BLK = 2048
_NB = (_N + _BLK - 1) // _BLK


def _views_np(n):
    phi = (np.sqrt(5) - 1) / 2
    i = np.arange(n)
    zi = (2 * i + 1) / n - 1
    r = np.sqrt(np.clip(1 - zi ** 2, 0.0, None))
    xi = r * np.cos(2 * i * np.pi * phi)
    yi = r * np.sin(2 * i * np.pi * phi)
    return np.stack([xi, yi, zi], -1).astype(np.float32)


_VIEWS = _views_np(_V)


def _head_a_body(x_ref, wg1_ref, bg1_ref, gam_ref, bet_ref, mu_ref, var_ref,
                 wg2_ref, bg2_ref, g_ref, m_ref, ft_ref):
    x = x_ref[0]
    h = jnp.dot(wg1_ref[...], x, preferred_element_type=jnp.float32)
    h = h + bg1_ref[0][:, None]
    h = gam_ref[0][:, None] * (h - mu_ref[0][:, None]) / jnp.sqrt(
        var_ref[0][:, None] + 1e-5) + bet_ref[0][:, None]
    h = jnp.maximum(h, 0.0)
    gr = jnp.dot(wg2_ref[...], h, preferred_element_type=jnp.float32)
    gr = gr + bg2_ref[0][:, None]
    s0 = gr[0:1, :]
    s1 = gr[1:2, :]
    g = gr[2:3, :]
    g_ref[...] = g.reshape(1, 1, -1)
    m_ref[...] = jnp.where((g > 0.1) & (s1 > s0), 1.0, 0.0).reshape(1, 1, -1)
    ft_ref[0] = x.T


def _stage_a(feats, wg1, bg1, gam, bet, mu, var, wg2, bg2):
    vspec = pl.BlockSpec((1, _C), lambda b, n: (0, 0))
    g, m, ft = pl.pallas_call(
        _head_a_body,
        grid=(_B, _NB),
        in_specs=[
            pl.BlockSpec((1, _C, _BLK), lambda b, n: (b, 0, n)),
            pl.BlockSpec((_C, _C), lambda b, n: (0, 0)),
            vspec, vspec, vspec, vspec, vspec,
            pl.BlockSpec((3, _C), lambda b, n: (0, 0)),
            pl.BlockSpec((1, 3), lambda b, n: (0, 0)),
        ],
        out_specs=[
            pl.BlockSpec((1, 1, _BLK), lambda b, n: (b, 0, n)),
            pl.BlockSpec((1, 1, _BLK), lambda b, n: (b, 0, n)),
            pl.BlockSpec((1, _BLK, _C), lambda b, n: (b, n, 0)),
        ],
        out_shape=[
            jax.ShapeDtypeStruct((_B, 1, _N), jnp.float32),
            jax.ShapeDtypeStruct((_B, 1, _N), jnp.float32),
            jax.ShapeDtypeStruct((_B, _N, _C), jnp.float32),
        ],
    )(feats, wg1, bg1.reshape(1, _C), gam.reshape(1, _C), bet.reshape(1, _C),
      mu.reshape(1, _C), var.reshape(1, _C), wg2, bg2.reshape(1, 3))
    return g[:, 0, :], m[:, 0, :], ft


def _fps_body(x_ref, y_ref, z_ref, pts_ref, mask_ref,
              inds_ref, gxyz_ref, fp2_ref, dist_ref):
    mask = mask_ref[...]
    iota = lax.broadcasted_iota(jnp.int32, (_B, _N), 1)
    mx = jnp.max(mask, axis=1, keepdims=True)
    first = jnp.min(jnp.where(mask == mx, iota, _N), axis=1).astype(jnp.int32)
    dist_ref[...] = jnp.where(mask > 0.0, 1e10, -1e10)

    def body(i, last):
        rows = []
        for b in range(_B):
            rows.append(pts_ref[pl.ds(last[b], 1), b, :])
        rows = jnp.concatenate(rows, axis=0)
        inds_ref[pl.ds(i, 1)] = last.reshape(1, 1, _B)
        gxyz_ref[pl.ds(i, 1)] = rows[:, 0:3].reshape(1, _B, 3)
        fp2_ref[pl.ds(i, 1)] = rows[:, 3:4].reshape(1, 1, _B)
        d = ((x_ref[...] - rows[:, 0:1]) ** 2
             + (y_ref[...] - rows[:, 1:2]) ** 2
             + (z_ref[...] - rows[:, 2:3]) ** 2)
        dd = jnp.minimum(dist_ref[...], d)
        dist_ref[...] = dd
        mv = jnp.max(dd, axis=1, keepdims=True)
        return jnp.min(jnp.where(dd == mv, iota, _N), axis=1).astype(jnp.int32)

    lax.fori_loop(0, _M, body, first)


def _stage_b(x, y, z, pts, mask):
    return pl.pallas_call(
        _fps_body,
        out_shape=[
            jax.ShapeDtypeStruct((_M, 1, _B), jnp.int32),
            jax.ShapeDtypeStruct((_M, _B, 3), jnp.float32),
            jax.ShapeDtypeStruct((_M, 1, _B), jnp.float32),
        ],
        scratch_shapes=[pltpu.VMEM((_B, _N), jnp.float32)],
    )(x, y, z, pts, mask)


_NW = 32
_RPW = (_B * _M) // _NW


def _sc_gather_body(table_hbm, idx_hbm, out_hbm, idx_v, rows_v, sem):
    wid = lax.axis_index("s") * 2 + lax.axis_index("c")
    base = wid * _RPW
    pltpu.sync_copy(idx_hbm.at[pl.ds(base, _RPW)], idx_v)
    boff = (wid // (_M // _RPW)) * _N
    for j in range(_RPW // 16):
        idx_v[pl.ds(j * 16, 16)] = idx_v[pl.ds(j * 16, 16)] + boff
    pltpu.async_copy(table_hbm.at[idx_v], rows_v, sem).wait()
    pltpu.sync_copy(rows_v, out_hbm.at[pl.ds(base, _RPW)])


@functools.partial(jax.jit, static_argnames=())
def _gather_rows(table, idx):
    mesh = plsc.VectorSubcoreMesh(core_axis_name="c", subcore_axis_name="s")
    f = functools.partial(
        pl.kernel,
        mesh=mesh,
        out_type=jax.ShapeDtypeStruct((_B * _M, _C), jnp.float32),
        scratch_types=[
            pltpu.VMEM((_RPW,), jnp.int32),
            pltpu.VMEM((_RPW, _C), jnp.float32),
            pltpu.SemaphoreType.DMA,
        ],
    )(_sc_gather_body)
    return f(table, idx)


def _head_d_body(ft_ref, w1_ref, b1_ref, gam_ref, bet_ref, mu_ref, var_ref,
                 w2_ref, b2_ref, views_ref, vp_ref, tvi_ref, rot_ref):
    ft = ft_ref[...]
    pre = lax.dot_general(ft, w1_ref[...], (((1,), (1,)), ((), ())),
                          preferred_element_type=jnp.float32)
    pre = pre + b1_ref[0][None, :]
    f = gam_ref[0][None, :] * (pre - mu_ref[0][None, :]) / jnp.sqrt(
        var_ref[0][None, :] + 1e-5) + bet_ref[0][None, :]
    f = jnp.maximum(f, 0.0)
    vp = lax.dot_general(f, w2_ref[...], (((1,), (1,)), ((), ())),
                         preferred_element_type=jnp.float32)
    vp = vp + b2_ref[0][None, :]
    vp_ref[...] = vp

    v0 = vp[:, 0:1]
    v1 = vp[:, 1:2]
    v2 = vp[:, 2:3]
    qn = vp / jnp.maximum(jnp.sqrt(v0 * v0 + v1 * v1 + v2 * v2), 1e-8)
    views = views_ref[...]
    w0 = views[:, 0:1]
    w1v = views[:, 1:2]
    w2v = views[:, 2:3]
    vn = views / jnp.maximum(jnp.sqrt(w0 * w0 + w1v * w1v + w2v * w2v), 1e-8)
    sim = lax.dot_general(qn, vn, (((1,), (1,)), ((), ())),
                          preferred_element_type=jnp.float32)
    iv = lax.broadcasted_iota(jnp.int32, sim.shape, 1)
    sm = jnp.max(sim, axis=1, keepdims=True)
    tvi_ref[...] = jnp.min(jnp.where(sim == sm, iv, _V), axis=1
                           ).astype(jnp.int32).reshape(-1, 1)

    t = -vp
    t0 = t[:, 0:1]
    t1 = t[:, 1:2]
    t2 = t[:, 2:3]
    zero = jnp.zeros_like(t0)
    ay = jnp.concatenate([-t1, t0, zero], axis=1)
    ny = jnp.sqrt(t1 * t1 + t0 * t0 + zero * zero)
    onevec = jnp.concatenate([zero, zero + 1.0, zero], axis=1)
    ay = jnp.where(ny == 0.0, onevec, ay)
    nx = jnp.sqrt(t0 * t0 + t1 * t1 + t2 * t2)
    axn = t / jnp.maximum(nx, 1e-12)
    a0 = ay[:, 0:1]
    a1 = ay[:, 1:2]
    a2 = ay[:, 2:3]
    nay = jnp.sqrt(a0 * a0 + a1 * a1 + a2 * a2)
    ayn = ay / jnp.maximum(nay, 1e-12)
    x0 = axn[:, 0:1]
    x1 = axn[:, 1:2]
    x2 = axn[:, 2:3]
    y0 = ayn[:, 0:1]
    y1 = ayn[:, 1:2]
    y2 = ayn[:, 2:3]
    z0 = x1 * y2 - x2 * y1
    z1 = x2 * y0 - x0 * y2
    z2 = x0 * y1 - x1 * y0
    rot_ref[...] = jnp.concatenate(
        [x0, y0, z0, x1, y1, z1, x2, y2, z2], axis=1)


def _stage_d(g_feat, w1, b1, gam, bet, mu, var, w2, b2, views):
    return pl.pallas_call(
        _head_d_body,
        out_shape=[
            jax.ShapeDtypeStruct((_B * _M, 3), jnp.float32),
            jax.ShapeDtypeStruct((_B * _M, 1), jnp.int32),
            jax.ShapeDtypeStruct((_B * _M, 9), jnp.float32),
        ],
    )(g_feat, w1, b1.reshape(1, _C), gam.reshape(1, _C), bet.reshape(1, _C),
      mu.reshape(1, _C), var.reshape(1, _C), w2, b2.reshape(1, 3), views)


def kernel(seed_xyz, seed_features, Wg1, bg1, gn_gamma, gn_beta, gn_mean,
           gn_var, Wg2, bg2, W1, b1, bn1_gamma, bn1_beta, bn1_mean, bn1_var,
           W2, b2):
    graspness, maskf, feat_t = _stage_a(
        seed_features, Wg1, bg1, gn_gamma, gn_beta, gn_mean, gn_var, Wg2, bg2)

    x = seed_xyz[:, :, 0]
    y = seed_xyz[:, :, 1]
    z = seed_xyz[:, :, 2]
    pts = jnp.transpose(
        jnp.concatenate([seed_xyz, graspness[:, :, None]], axis=2), (1, 0, 2))

    inds_m1b, gxyz_mb3, fp2_m1b = _stage_b(x, y, z, pts, maskf)
    inds = jnp.transpose(inds_m1b[:, 0, :], (1, 0))
    g_xyz = jnp.transpose(gxyz_mb3, (1, 0, 2))
    fp2_graspness = jnp.transpose(fp2_m1b[:, 0, :], (1, 0))

    table = feat_t.reshape(_B * _N, _C)
    g_feat = _gather_rows(table, inds.reshape(_B * _M))

    vp, tvi, rot = _stage_d(g_feat, W1, b1, bn1_gamma, bn1_beta, bn1_mean,
                            bn1_var, W2, b2, jnp.asarray(_VIEWS))
    vp_xyz = vp.reshape(_B, _M, 3)
    top_view_inds = tvi.reshape(_B, _M)
    vp_rot = rot.reshape(_B, _M, 3, 3)
    return vp_xyz, top_view_inds, vp_rot, g_xyz, fp2_graspness
